# Initial kernel scaffold; baseline (speedup 1.0000x reference)
import jax
import jax.numpy as jnp
from jax.experimental import pallas as pl


def kernel(x, W1, b1, Wc1, bc1, Wc2, bc2, W2, b2):
    raise NotImplementedError("write your pallas kernel here")


# speedup vs baseline: 620.9974x; 620.9974x over previous
import jax
import jax.numpy as jnp
from jax.experimental import pallas as pl

N = 1024
IN_CH = 1024
HID = 64
OUT_CH = 64


def _gcn_body(x_ref, W1_ref, b1_ref, Wc1_ref, bc1_ref, Wc2_ref, bc2_ref,
              h2_ref):
    x = x_ref[...]
    ada = jnp.dot(x, W1_ref[...], preferred_element_type=jnp.float32)
    ada = ada + b1_ref[...]
    m = jnp.where(ada != 0.0, 1.0, 0.0)
    ones_col = jnp.ones((N, 1), dtype=jnp.float32)
    deg = jax.lax.dot_general(m, ones_col, (((0,), (0,)), ((), ())),
                              preferred_element_type=jnp.float32) + 1.0
    dinv = jax.lax.rsqrt(deg)

    xw = jnp.dot(x, Wc1_ref[...], preferred_element_type=jnp.float32)
    y = xw * dinv
    z = jax.lax.dot_general(m, y, (((0,), (0,)), ((), ())),
                            preferred_element_type=jnp.float32) + y
    h1 = jnp.maximum(z * dinv + bc1_ref[...], 0.0)

    xw2 = jnp.dot(h1, Wc2_ref[...], preferred_element_type=jnp.float32)
    y2 = xw2 * dinv
    z2 = jax.lax.dot_general(m, y2, (((0,), (0,)), ((), ())),
                             preferred_element_type=jnp.float32) + y2
    h2_ref[...] = z2 * dinv + bc2_ref[...]


def _readout_body(v_ref, W2_ref, b2_ref, o_ref):
    o_ref[...] = jnp.dot(v_ref[...], W2_ref[...],
                         preferred_element_type=jnp.float32) + b2_ref[...]


def kernel(x, W1, b1, Wc1, bc1, Wc2, bc2, W2, b2):
    h2 = pl.pallas_call(
        _gcn_body,
        out_shape=jax.ShapeDtypeStruct((N, OUT_CH), jnp.float32),
    )(x, W1, b1.reshape(1, IN_CH), Wc1, bc1.reshape(1, HID), Wc2,
      bc2.reshape(1, OUT_CH))
    v = h2.reshape(1, N * OUT_CH)
    out = pl.pallas_call(
        _readout_body,
        out_shape=jax.ShapeDtypeStruct((1, OUT_CH), jnp.float32),
    )(v, W2, b2.reshape(1, OUT_CH))
    return out.reshape(OUT_CH)
